# baseline (speedup 1.0000x reference)
import functools

import jax
import jax.numpy as jnp
from jax import lax
from jax.experimental import pallas as pl
from jax.experimental.pallas import tpu as pltpu

N = 10000
E = 640000
K = 16
HID = 64
OUT = 64
IN = 3


def _mlp2_body(m_ref, wa_ref, ba_ref, wb_ref, bb_ref, o_ref):
    m = m_ref[...]
    h = jnp.maximum(jnp.dot(m, wa_ref[...], preferred_element_type=jnp.float32)
                    + ba_ref[...], 0.0)
    h = jnp.maximum(jnp.dot(h, wb_ref[...], preferred_element_type=jnp.float32)
                    + bb_ref[...], 0.0)
    o_ref[...] = h


def _mlp2(m, wa, ba, wb, bb, block):
    mm, d = m.shape
    dmid = wa.shape[1]
    dout = wb.shape[1]
    return pl.pallas_call(
        _mlp2_body,
        grid=(mm // block,),
        in_specs=[
            pl.BlockSpec((block, d), lambda i: (i, 0)),
            pl.BlockSpec((d, dmid), lambda i: (0, 0)),
            pl.BlockSpec((1, dmid), lambda i: (0, 0)),
            pl.BlockSpec((dmid, dout), lambda i: (0, 0)),
            pl.BlockSpec((1, dout), lambda i: (0, 0)),
        ],
        out_specs=pl.BlockSpec((block, dout), lambda i: (i, 0)),
        out_shape=jax.ShapeDtypeStruct((mm, dout), jnp.float32),
    )(m, wa, ba.reshape(1, dmid), wb, bb.reshape(1, dout))


def _mlp2_max_body(m_ref, wa_ref, ba_ref, wb_ref, bb_ref, o_ref, *, bn):
    m = m_ref[...]
    h = jnp.maximum(jnp.dot(m, wa_ref[...], preferred_element_type=jnp.float32)
                    + ba_ref[...], 0.0)
    h = jnp.maximum(jnp.dot(h, wb_ref[...], preferred_element_type=jnp.float32)
                    + bb_ref[...], 0.0)
    o_ref[...] = jnp.max(h.reshape(bn, K, -1), axis=1)


def _mlp2_max(m, wa, ba, wb, bb, bn):
    n = m.shape[0] // K
    d = m.shape[1]
    dmid = wa.shape[1]
    dout = wb.shape[1]
    return pl.pallas_call(
        functools.partial(_mlp2_max_body, bn=bn),
        grid=(n // bn,),
        in_specs=[
            pl.BlockSpec((bn * K, d), lambda i: (i, 0)),
            pl.BlockSpec((d, dmid), lambda i: (0, 0)),
            pl.BlockSpec((1, dmid), lambda i: (0, 0)),
            pl.BlockSpec((dmid, dout), lambda i: (0, 0)),
            pl.BlockSpec((1, dout), lambda i: (0, 0)),
        ],
        out_specs=pl.BlockSpec((bn, dout), lambda i: (i, 0)),
        out_shape=jax.ShapeDtypeStruct((n, dout), jnp.float32),
    )(m, wa, ba.reshape(1, dmid), wb, bb.reshape(1, dout))


def kernel(x, edge_index, W1, b1, W2, b2, W3, b3, W4, b4):
    src = edge_index[0]
    dst = edge_index[1]

    xi = x[dst]
    xj = x[src]
    msg = jnp.concatenate([xi, xj - xi], axis=-1)
    h = _mlp2(msg, W1, b1, W2, b2, block=2560)
    x1 = jax.ops.segment_max(h, dst, num_segments=N)
    x1 = jnp.where(jnp.isfinite(x1), x1, 0.0)

    sq = jnp.sum(x1 * x1, axis=1)
    def body(q):
        d2 = jnp.sum(q * q, axis=1)[:, None] - 2.0 * (q @ x1.T) + sq[None, :]
        _, idx = jax.lax.top_k(-d2, K)
        return idx
    qs = x1.reshape(N // 1000, 1000, HID)
    idx = jax.lax.map(body, qs).reshape(N, K)

    nj = x1[idx]
    ni = jnp.broadcast_to(x1[:, None, :], nj.shape)
    msg2 = jnp.concatenate([ni, nj - ni], axis=-1).reshape(N * K, 2 * HID)
    out = _mlp2_max(msg2, W3, b3, W4, b4, bn=80)
    return out
